# baseline (speedup 1.0000x reference)
import functools

import jax
import jax.numpy as jnp
from jax import lax
from jax.experimental import pallas as pl
from jax.experimental.pallas import tpu as pltpu
from jax.experimental.pallas import tpu_sc as plsc

B = 1024
FEAT = 64
K = 512
TAU = 0.07
T = 4.0
NUM_CORES = 2
NUM_SUBCORES = 16
NW = NUM_CORES * NUM_SUBCORES
B_PER_W = B // NW
CHUNK = 64
NCHUNK = K // CHUNK
NBLK = CHUNK // 16
NEG_INF = -1e30


def _sc_body(idx_hbm, emb0_hbm, emb1_hbm, mem0_hbm, mem1_hbm,
             d00_hbm, d01_hbm, d10_hbm, d11_hbm,
             idx_v, e0_v, e1_v, buf0, buf1,
             o00_v, o01_v, o10_v, o11_v, sem0, sem1):
    wid = lax.axis_index("s") * NUM_CORES + lax.axis_index("c")
    row_ids = [blk * 16 + lax.iota(jnp.int32, 16) for blk in range(NBLK)]

    def batch_body(i, carry):
        b = wid * B_PER_W + i
        pltpu.sync_copy(idx_hbm.at[b], idx_v)
        pltpu.sync_copy(emb0_hbm.at[b], e0_v)
        pltpu.sync_copy(emb1_hbm.at[b], e1_v)

        def chunk_body(c, carry2):
            base = pl.multiple_of(c * CHUNK, CHUNK)
            idx_sl = idx_v.at[pl.ds(base, CHUNK)]
            cp0 = pltpu.async_copy(mem0_hbm.at[idx_sl], buf0, sem0)
            cp1 = pltpu.async_copy(mem1_hbm.at[idx_sl], buf1, sem1)
            cp0.wait()
            cp1.wait()
            acc00 = [jnp.zeros((16,), jnp.float32) for _ in range(NBLK)]
            acc01 = [jnp.zeros((16,), jnp.float32) for _ in range(NBLK)]
            acc10 = [jnp.zeros((16,), jnp.float32) for _ in range(NBLK)]
            acc11 = [jnp.zeros((16,), jnp.float32) for _ in range(NBLK)]
            for dv in range(FEAT // 16):
                ev0 = e0_v[pl.ds(dv * 16, 16)]
                ev1 = e1_v[pl.ds(dv * 16, 16)]
                for lane in range(16):
                    d = dv * 16 + lane
                    e0 = ev0[lane]
                    e1 = ev1[lane]
                    col = jnp.full((16,), d, jnp.int32)
                    for blk in range(NBLK):
                        v0 = plsc.load_gather(buf0, [row_ids[blk], col])
                        v1 = plsc.load_gather(buf1, [row_ids[blk], col])
                        acc00[blk] = acc00[blk] + v0 * e0
                        acc01[blk] = acc01[blk] + v0 * e1
                        acc10[blk] = acc10[blk] + v1 * e0
                        acc11[blk] = acc11[blk] + v1 * e1
            for blk in range(NBLK):
                sl = pl.ds(base + blk * 16, 16)
                o00_v[sl] = acc00[blk]
                o01_v[sl] = acc01[blk]
                o10_v[sl] = acc10[blk]
                o11_v[sl] = acc11[blk]
            return 0

        lax.fori_loop(0, NCHUNK, chunk_body, 0)
        pltpu.sync_copy(o00_v, d00_hbm.at[b])
        pltpu.sync_copy(o01_v, d01_hbm.at[b])
        pltpu.sync_copy(o10_v, d10_hbm.at[b])
        pltpu.sync_copy(o11_v, d11_hbm.at[b])
        return 0

    lax.fori_loop(0, B_PER_W, batch_body, 0)


@functools.cache
def _make_sc_call():
    return pl.kernel(
        _sc_body,
        out_type=[jax.ShapeDtypeStruct((B, K), jnp.float32) for _ in range(4)],
        mesh=plsc.VectorSubcoreMesh(core_axis_name="c", subcore_axis_name="s",
                                    num_cores=NUM_CORES, num_subcores=NUM_SUBCORES),
        scratch_types=[
            pltpu.VMEM((K,), jnp.int32),
            pltpu.VMEM((FEAT,), jnp.float32),
            pltpu.VMEM((FEAT,), jnp.float32),
            pltpu.VMEM((CHUNK, FEAT), jnp.float32),
            pltpu.VMEM((CHUNK, FEAT), jnp.float32),
            pltpu.VMEM((K,), jnp.float32),
            pltpu.VMEM((K,), jnp.float32),
            pltpu.VMEM((K,), jnp.float32),
            pltpu.VMEM((K,), jnp.float32),
            pltpu.SemaphoreType.DMA,
            pltpu.SemaphoreType.DMA,
        ],
        compiler_params=pltpu.CompilerParams(needs_layout_passes=False,
                                             use_tc_tiling_on_sc=False),
    )


def _lse(x):
    m = jnp.max(x, axis=1, keepdims=True)
    return m + jnp.log(jnp.sum(jnp.exp(x - m), axis=1, keepdims=True))


def _klsum(ys, yt):
    s = ys / T
    t = yt / T
    logps = s - _lse(s)
    logpt = t - _lse(t)
    pt = jnp.exp(logpt)
    return jnp.sum(pt * (logpt - logps)) / B * (T * T)


def _tc_body(d00_ref, d01_ref, d10_ref, d11_ref,
             vcl_ref, svcl_ref, icl_ref, sicl_ref):
    inv_tau = 1.0 / TAU
    A = d01_ref[...] * inv_tau
    C = d10_ref[...] * inv_tau
    U = d00_ref[...] * inv_tau
    V = d11_ref[...] * inv_tau
    col = lax.broadcasted_iota(jnp.int32, (B, K), 1)

    def icl_term(X):
        pos = jnp.sum(jnp.where(col < 2, X, 0.0), axis=1, keepdims=True) * 0.5
        return jnp.mean(_lse(X) - pos)

    icl_ref[0, 0] = icl_term(A) + icl_term(C)
    sicl_ref[0, 0] = _klsum(A, C) + _klsum(C, A)

    Um = jnp.where(col >= 1, U, NEG_INF)
    Vm = jnp.where(col >= 1, V, NEG_INF)

    def vcl_term(Xm, X):
        pos = jnp.sum(jnp.where(col == 1, X, 0.0), axis=1, keepdims=True)
        return jnp.mean(_lse(Xm) - pos)

    vcl_ref[0, 0] = vcl_term(Um, U) + vcl_term(Vm, V)
    svcl_ref[0, 0] = _klsum(Um, Vm) + _klsum(Vm, Um)


@functools.cache
def _make_tc_call():
    return pl.pallas_call(
        _tc_body,
        out_shape=[jax.ShapeDtypeStruct((1, 1), jnp.float32) for _ in range(4)],
        out_specs=[pl.BlockSpec(memory_space=pltpu.SMEM) for _ in range(4)],
    )


def kernel(emb0, emb1, pos_idx, neg_idx, memory_0, memory_1):
    idx = jnp.concatenate([pos_idx, neg_idx], axis=1)
    d00, d01, d10, d11 = _make_sc_call()(idx, emb0, emb1, memory_0, memory_1)
    vcl, svcl, icl, sicl = _make_tc_call()(d00, d01, d10, d11)
    return (vcl[0, 0], svcl[0, 0], icl[0, 0], sicl[0, 0])
